# Initial kernel scaffold; baseline (speedup 1.0000x reference)
import jax
import jax.numpy as jnp
from jax.experimental import pallas as pl


def kernel(x, edge_index, W1, a1_src, a1_dst, b1, W2, a2_src, a2_dst, b2, W3, a3_src, a3_dst, b3):
    raise NotImplementedError("write your pallas kernel here")


# speedup vs baseline: 29.8515x; 29.8515x over previous
import functools

import jax
import jax.numpy as jnp
from jax import lax
from jax.experimental import pallas as pl
from jax.experimental.pallas import tpu as pltpu
from jax.experimental.pallas import tpu_sc as plsc

_F32 = jnp.float32
_I32 = jnp.int32

_NW = 32
_B = 128
_ZR = 64
_BN = 1024


def _pre_body(x_ref, W_ref, PsLo_ref, PsHi_ref, PdLo_ref, PdHi_ref,
              mlo_ref, mhi_ref, slo_ref, shi_ref, alo_ref, ahi_ref):
    h = jnp.dot(x_ref[...], W_ref[...], preferred_element_type=_F32)
    mlo_ref[...] = h[:, :64]
    mhi_ref[...] = h[:, 64:]
    slo_ref[...] = jnp.dot(h, PsLo_ref[...], preferred_element_type=_F32)
    shi_ref[...] = jnp.dot(h, PsHi_ref[...], preferred_element_type=_F32)
    alo_ref[...] = jnp.dot(h, PdLo_ref[...], preferred_element_type=_F32)
    ahi_ref[...] = jnp.dot(h, PdHi_ref[...], preferred_element_type=_F32)


def _mid2_body(amlo_ref, amhi_ref, ae_ref, S_ref, b_ref, W_ref, Ps_ref,
               Pd_ref, m_ref, s_ref, a_ref):
    num = jnp.concatenate(
        [amlo_ref[0] + amlo_ref[1], amhi_ref[0] + amhi_ref[1]], axis=1)
    den = ae_ref[0] + ae_ref[1]
    recip = 1.0 / (den[:, :8] + 1e-16)
    xn = num * jnp.dot(recip, S_ref[...], preferred_element_type=_F32) + b_ref[...]
    xn = jnp.where(xn > 0, xn, jnp.exp(xn) - 1.0)
    h = jnp.dot(xn, W_ref[...], preferred_element_type=_F32)
    m_ref[...] = h
    s_ref[...] = jnp.dot(h, Ps_ref[...], preferred_element_type=_F32)
    a_ref[...] = jnp.dot(h, Pd_ref[...], preferred_element_type=_F32)


def _mid3_body(am_ref, ae_ref, S_ref, b_ref, W_ref, Ps_ref, Pd_ref,
               m_ref, s_ref, a_ref):
    num = am_ref[0] + am_ref[1]
    den = ae_ref[0] + ae_ref[1]
    recip = 1.0 / (den[:, :8] + 1e-16)
    xn = num * jnp.dot(recip, S_ref[...], preferred_element_type=_F32) + b_ref[...]
    xn = jnp.where(xn > 0, xn, jnp.exp(xn) - 1.0)
    h = jnp.dot(xn, W_ref[...], preferred_element_type=_F32)
    m_ref[...] = h
    s_ref[...] = jnp.dot(h, Ps_ref[...], preferred_element_type=_F32)
    a_ref[...] = jnp.dot(h, Pd_ref[...], preferred_element_type=_F32)


def _final_body(am_ref, ae_ref, b_ref, out_ref):
    num = am_ref[0] + am_ref[1]
    den = ae_ref[0] + ae_ref[1]
    z = num / (den[:, 0:1] + 1e-16) + b_ref[...]
    col = lax.broadcasted_iota(_I32, z.shape, 1)
    valid = col < 10
    zm = jnp.where(valid, z, -1e30)
    m = jnp.max(zm, axis=1, keepdims=True)
    ssum = jnp.sum(jnp.where(valid, jnp.exp(z - m), 0.0), axis=1, keepdims=True)
    out_ref[...] = zm - m - jnp.log(ssum)


def _full_spec(shape):
    nd = len(shape)
    return pl.BlockSpec(shape, lambda i, _n=nd: (0,) * _n)


def _row_spec(w):
    return pl.BlockSpec((_BN, w), lambda i: (i, 0))


def _acc_spec(w):
    return pl.BlockSpec((2, _BN, w), lambda i: (0, i, 0))


def _make_sc(n_pad, wm, with_e, ept):
    wa = wm + 16 if with_e else wm
    nch = wm // 16
    nchunks = ept // _B
    rows_per_tile = n_pad // 16
    mesh = plsc.VectorSubcoreMesh(core_axis_name="c", subcore_axis_name="s")

    out_type = [jax.ShapeDtypeStruct((2, n_pad, wm), _F32)]
    scratch = [
        pltpu.VMEM((_B,), _I32),
        pltpu.VMEM((_B,), _I32),
        pltpu.VMEM((_B, wm), _F32),
        pltpu.VMEM((_B, wa), _F32),
        pltpu.VMEM((_B, wa), _F32),
        pltpu.VMEM((_ZR, wm), _F32),
        pltpu.VMEM_SHARED((n_pad, wm), _F32),
        pltpu.SemaphoreType.DMA,
    ]
    if with_e:
        out_type.append(jax.ShapeDtypeStruct((2, n_pad, 16), _F32))
        scratch += [
            pltpu.VMEM((_B, 16), _F32),
            pltpu.VMEM((_ZR, 16), _F32),
            pltpu.VMEM_SHARED((n_pad, 16), _F32),
        ]

    @functools.partial(
        pl.kernel,
        out_type=tuple(out_type),
        mesh=mesh,
        compiler_params=pltpu.CompilerParams(use_tc_tiling_on_sc=False),
        scratch_types=scratch,
    )
    def sc_kernel(m_hbm, s_hbm, a_hbm, src_hbm, dst_hbm, *rest):
        if with_e:
            (outm_hbm, oute_hbm, src_v, dst_v, msg_v, as_v, ad_v, zm_v,
             accm_sh, sem, e_v, ze_v, acce_sh) = rest
        else:
            (outm_hbm, src_v, dst_v, msg_v, as_v, ad_v, zm_v,
             accm_sh, sem) = rest
        c = lax.axis_index("c")
        s = lax.axis_index("s")
        wid = s * 2 + c
        zvec = jnp.zeros((16,), _F32)

        def zero_buf(r, carry):
            for k in range(nch):
                zm_v[r, pl.ds(k * 16, 16)] = zvec
            if with_e:
                ze_v[r, pl.ds(0, 16)] = zvec
            return carry
        lax.fori_loop(0, _ZR, zero_buf, 0)

        def zero_acc(i, carry):
            row = s * rows_per_tile + i * _ZR
            pltpu.sync_copy(zm_v, accm_sh.at[pl.ds(row, _ZR)])
            if with_e:
                pltpu.sync_copy(ze_v, acce_sh.at[pl.ds(row, _ZR)])
            return carry
        lax.fori_loop(0, rows_per_tile // _ZR, zero_acc, 0)
        plsc.subcore_barrier()

        base_e = wid * ept

        def chunk(chk, carry):
            eb = base_e + chk * _B
            pltpu.sync_copy(src_hbm.at[pl.ds(eb, _B)], src_v)
            pltpu.sync_copy(dst_hbm.at[pl.ds(eb, _B)], dst_v)
            cp1 = pltpu.async_copy(m_hbm.at[src_v], msg_v, sem)
            cp2 = pltpu.async_copy(s_hbm.at[src_v], as_v, sem)
            cp3 = pltpu.async_copy(a_hbm.at[dst_v], ad_v, sem)
            cp1.wait()
            cp2.wait()
            cp3.wait()

            def edge(b, cr):
                for k in range(nch):
                    sl = pl.ds(k * 16, 16)
                    al = as_v[b, sl] + ad_v[b, sl]
                    e16 = jnp.exp(jnp.maximum(al, 0.2 * al))
                    msg_v[b, sl] = msg_v[b, sl] * e16
                if with_e:
                    sc_ = as_v[b, pl.ds(wm, 16)] + ad_v[b, pl.ds(wm, 16)]
                    e_v[b, pl.ds(0, 16)] = jnp.exp(jnp.maximum(sc_, 0.2 * sc_))
                return cr
            lax.fori_loop(0, _B, edge, 0)

            pltpu.sync_copy(msg_v, accm_sh.at[dst_v], add=True)
            if with_e:
                pltpu.sync_copy(e_v, acce_sh.at[dst_v], add=True)
            return carry
        lax.fori_loop(0, nchunks, chunk, 0)

        plsc.subcore_barrier()
        row0 = s * rows_per_tile
        pltpu.sync_copy(accm_sh.at[pl.ds(row0, rows_per_tile)],
                        outm_hbm.at[c, pl.ds(row0, rows_per_tile)])
        if with_e:
            pltpu.sync_copy(acce_sh.at[pl.ds(row0, rows_per_tile)],
                            oute_hbm.at[c, pl.ds(row0, rows_per_tile)])

    return sc_kernel


def _exp_compact(a):
    hh, cc = a.shape
    hc = hh * cc
    eye = jnp.eye(hh, dtype=_F32)
    Ac = (a[:, :, None] * eye[:, None, :]).reshape(hc, hh)
    rep = jnp.repeat(eye, cc, axis=1)
    E = jnp.dot(Ac, rep)
    Ac16 = jnp.pad(Ac, ((0, 0), (0, 16 - hh)))
    return E, Ac16


def kernel(x, edge_index, W1, a1_src, a1_dst, b1, W2, a2_src, a2_dst, b2,
           W3, a3_src, a3_dst, b3):
    n = x.shape[0]
    loops = jnp.arange(n, dtype=edge_index.dtype)
    src = jnp.concatenate([edge_index[0], loops]).astype(_I32)
    dst = jnp.concatenate([edge_index[1], loops]).astype(_I32)
    e_tot = src.shape[0]
    ept = -(-e_tot // (_NW * _B)) * _B
    e_pad = ept * _NW
    if e_pad > e_tot:
        fill = jnp.full((e_pad - e_tot,), n, _I32)
        src = jnp.concatenate([src, fill])
        dst = jnp.concatenate([dst, fill])
    n_pad = -(-(n + 1) // 1024) * 1024
    xp = jnp.pad(x, ((0, n_pad - n), (0, 0)))

    Es1, Ac1s = _exp_compact(a1_src)
    Ed1, Ac1d = _exp_compact(a1_dst)
    PsLo1 = jnp.concatenate([Es1[:, :64], Ac1s], axis=1)
    PsHi1 = Es1[:, 64:]
    PdLo1 = jnp.concatenate([Ed1[:, :64], Ac1d], axis=1)
    PdHi1 = Ed1[:, 64:]
    Es2, Ac2s = _exp_compact(a2_src)
    Ed2, Ac2d = _exp_compact(a2_dst)
    Ps2 = jnp.concatenate([Es2, Ac2s], axis=1)
    Pd2 = jnp.concatenate([Ed2, Ac2d], axis=1)
    Es3, Ac3s = _exp_compact(a3_src)
    Ed3, Ac3d = _exp_compact(a3_dst)
    Ps3 = jnp.pad(jnp.concatenate([jnp.pad(Es3, ((0, 0), (0, 6))), Ac3s],
                                  axis=1), ((0, 6), (0, 0)))
    Pd3 = jnp.pad(jnp.concatenate([jnp.pad(Ed3, ((0, 0), (0, 6))), Ac3d],
                                  axis=1), ((0, 6), (0, 0)))
    W3p = jnp.pad(W3, ((0, 0), (0, 6)))
    S1 = jnp.repeat(jnp.eye(8, dtype=_F32), 16, axis=1)
    S2 = jnp.repeat(jnp.eye(8, dtype=_F32), 8, axis=1)
    b1r = b1.reshape(1, 128)
    b2r = b2.reshape(1, 64)
    b3r = jnp.pad(b3, (0, 6)).reshape(1, 16)

    grid = (n_pad // _BN,)

    mlo, mhi, slo, shi, alo, ahi = pl.pallas_call(
        _pre_body,
        grid=grid,
        in_specs=[_row_spec(128), _full_spec((128, 128)),
                  _full_spec((128, 80)), _full_spec((128, 64)),
                  _full_spec((128, 80)), _full_spec((128, 64))],
        out_specs=(_row_spec(64), _row_spec(64), _row_spec(80),
                   _row_spec(64), _row_spec(80), _row_spec(64)),
        out_shape=(
            jax.ShapeDtypeStruct((n_pad, 64), _F32),
            jax.ShapeDtypeStruct((n_pad, 64), _F32),
            jax.ShapeDtypeStruct((n_pad, 80), _F32),
            jax.ShapeDtypeStruct((n_pad, 64), _F32),
            jax.ShapeDtypeStruct((n_pad, 80), _F32),
            jax.ShapeDtypeStruct((n_pad, 64), _F32),
        ),
    )(xp, W1, PsLo1, PsHi1, PdLo1, PdHi1)

    amlo, ae1 = _make_sc(n_pad, 64, True, ept)(mlo, slo, alo, src, dst)
    (amhi,) = _make_sc(n_pad, 64, False, ept)(mhi, shi, ahi, src, dst)

    m2, s2, a2 = pl.pallas_call(
        _mid2_body,
        grid=grid,
        in_specs=[_acc_spec(64), _acc_spec(64), _acc_spec(16),
                  _full_spec((8, 128)), _full_spec((1, 128)),
                  _full_spec((128, 64)), _full_spec((64, 80)),
                  _full_spec((64, 80))],
        out_specs=(_row_spec(64), _row_spec(80), _row_spec(80)),
        out_shape=(
            jax.ShapeDtypeStruct((n_pad, 64), _F32),
            jax.ShapeDtypeStruct((n_pad, 80), _F32),
            jax.ShapeDtypeStruct((n_pad, 80), _F32),
        ),
    )(amlo, amhi, ae1, S1, b1r, W2, Ps2, Pd2)

    am2, ae2 = _make_sc(n_pad, 64, True, ept)(m2, s2, a2, src, dst)

    m3, s3, a3 = pl.pallas_call(
        _mid3_body,
        grid=grid,
        in_specs=[_acc_spec(64), _acc_spec(16),
                  _full_spec((8, 64)), _full_spec((1, 64)),
                  _full_spec((64, 16)), _full_spec((16, 32)),
                  _full_spec((16, 32))],
        out_specs=(_row_spec(16), _row_spec(32), _row_spec(32)),
        out_shape=(
            jax.ShapeDtypeStruct((n_pad, 16), _F32),
            jax.ShapeDtypeStruct((n_pad, 32), _F32),
            jax.ShapeDtypeStruct((n_pad, 32), _F32),
        ),
    )(am2, ae2, S2, b2r, W3p, Ps3, Pd3)

    am3, ae3 = _make_sc(n_pad, 16, True, ept)(m3, s3, a3, src, dst)

    out = pl.pallas_call(
        _final_body,
        grid=grid,
        in_specs=[_acc_spec(16), _acc_spec(16), _full_spec((1, 16))],
        out_specs=pl.BlockSpec((_BN, 16), lambda i: (i, 0)),
        out_shape=jax.ShapeDtypeStruct((n_pad, 16), _F32),
    )(am3, ae3, b3r)
    return out[:n, :10]


# speedup vs baseline: 52.4271x; 1.7563x over previous
import functools

import jax
import jax.numpy as jnp
from jax import lax
from jax.experimental import pallas as pl
from jax.experimental.pallas import tpu as pltpu
from jax.experimental.pallas import tpu_sc as plsc

_F32 = jnp.float32
_I32 = jnp.int32

_NW = 32
_B = 128
_ZR = 64
_BN = 1024


def _pre_body(x_ref, W_ref, Acs_ref, Acd_ref,
              glo_ref, ghi_ref, ad_ref):
    h = jnp.dot(x_ref[...], W_ref[...], preferred_element_type=_F32)
    asrc = jnp.dot(h, Acs_ref[...], preferred_element_type=_F32)
    glo_ref[...] = jnp.concatenate([h[:, :64], asrc], axis=1)
    ghi_ref[...] = jnp.concatenate([h[:, 64:], asrc], axis=1)
    ad_ref[...] = jnp.dot(h, Acd_ref[...], preferred_element_type=_F32)


def _mid2_body(accA_ref, accB_ref, S_ref, b_ref, W_ref, Acs_ref, Acd_ref,
               g_ref, ad_ref):
    accA = accA_ref[0] + accA_ref[1]
    accB = accB_ref[0] + accB_ref[1]
    num = jnp.concatenate([accA[:, :64], accB[:, :64]], axis=1)
    recip = 1.0 / (accA[:, 64:72] + 1e-16)
    xn = num * jnp.dot(recip, S_ref[...], preferred_element_type=_F32) + b_ref[...]
    xn = jnp.where(xn > 0, xn, jnp.exp(xn) - 1.0)
    h = jnp.dot(xn, W_ref[...], preferred_element_type=_F32)
    g_ref[...] = jnp.concatenate(
        [h, jnp.dot(h, Acs_ref[...], preferred_element_type=_F32)], axis=1)
    ad_ref[...] = jnp.dot(h, Acd_ref[...], preferred_element_type=_F32)


def _mid3_body(acc_ref, S_ref, b_ref, W_ref, Acs_ref, Acd_ref,
               g_ref, ad_ref):
    acc = acc_ref[0] + acc_ref[1]
    num = acc[:, :64]
    recip = 1.0 / (acc[:, 64:72] + 1e-16)
    xn = num * jnp.dot(recip, S_ref[...], preferred_element_type=_F32) + b_ref[...]
    xn = jnp.where(xn > 0, xn, jnp.exp(xn) - 1.0)
    h = jnp.dot(xn, W_ref[...], preferred_element_type=_F32)
    g_ref[...] = jnp.concatenate(
        [h, jnp.dot(h, Acs_ref[...], preferred_element_type=_F32)], axis=1)
    ad_ref[...] = jnp.dot(h, Acd_ref[...], preferred_element_type=_F32)


def _final_body(acc_ref, b_ref, out_ref):
    acc = acc_ref[0] + acc_ref[1]
    z = acc[:, :16] / (acc[:, 16:17] + 1e-16) + b_ref[...]
    col = lax.broadcasted_iota(_I32, z.shape, 1)
    valid = col < 10
    zm = jnp.where(valid, z, -1e30)
    m = jnp.max(zm, axis=1, keepdims=True)
    ssum = jnp.sum(jnp.where(valid, jnp.exp(z - m), 0.0), axis=1, keepdims=True)
    out_ref[...] = zm - m - jnp.log(ssum)


def _full_spec(shape):
    nd = len(shape)
    return pl.BlockSpec(shape, lambda i, _n=nd: (0,) * _n)


def _row_spec(w):
    return pl.BlockSpec((_BN, w), lambda i: (i, 0))


def _acc_spec(w):
    return pl.BlockSpec((2, _BN, w), lambda i: (0, i, 0))


def _make_sc(n_pad, wm, ch, hoff, ept):
    wr = wm + 16
    nch = wm // 16
    nchunks = ept // _B
    rows_per_tile = n_pad // 16
    mesh = plsc.VectorSubcoreMesh(core_axis_name="c", subcore_axis_name="s")

    @functools.partial(
        pl.kernel,
        out_type=jax.ShapeDtypeStruct((2, n_pad, wr), _F32),
        mesh=mesh,
        compiler_params=pltpu.CompilerParams(use_tc_tiling_on_sc=False),
        scratch_types=[
            pltpu.VMEM((_B,), _I32),
            pltpu.VMEM((_B,), _I32),
            pltpu.VMEM((_B, wr), _F32),
            pltpu.VMEM((_B, 16), _F32),
            pltpu.VMEM((_ZR, wr), _F32),
            pltpu.VMEM_SHARED((n_pad, wr), _F32),
            pltpu.SemaphoreType.DMA,
        ],
    )
    def sc_kernel(g_hbm, ad_hbm, src_hbm, dst_hbm, out_hbm,
                  src_v, dst_v, g_v, ad_v, z_v, acc_sh, sem):
        c = lax.axis_index("c")
        s = lax.axis_index("s")
        wid = s * 2 + c
        zvec = jnp.zeros((16,), _F32)

        def zero_buf(r, carry):
            for k in range(wr // 16):
                z_v[r, pl.ds(k * 16, 16)] = zvec
            return carry
        lax.fori_loop(0, _ZR, zero_buf, 0)

        def zero_acc(i, carry):
            pltpu.sync_copy(z_v, acc_sh.at[pl.ds(s * rows_per_tile + i * _ZR, _ZR)])
            return carry
        lax.fori_loop(0, rows_per_tile // _ZR, zero_acc, 0)
        plsc.subcore_barrier()

        base_e = wid * ept
        lanes = lax.iota(_I32, 16)

        def chunk(chk, carry):
            eb = base_e + chk * _B
            pltpu.sync_copy(src_hbm.at[pl.ds(eb, _B)], src_v)
            pltpu.sync_copy(dst_hbm.at[pl.ds(eb, _B)], dst_v)
            cp1 = pltpu.async_copy(g_hbm.at[src_v], g_v, sem)
            cp2 = pltpu.async_copy(ad_hbm.at[dst_v], ad_v, sem)
            cp1.wait()
            cp2.wait()

            def edge(b, cr):
                al = g_v[b, pl.ds(wm, 16)] + ad_v[b, pl.ds(0, 16)]
                e16 = jnp.exp(jnp.maximum(al, 0.2 * al))
                g_v[b, pl.ds(wm, 16)] = e16
                for k in range(nch):
                    sl = pl.ds(k * 16, 16)
                    if ch == 16:
                        ev = e16[hoff + k]
                        g_v[b, sl] = g_v[b, sl] * ev
                    else:
                        ev = jnp.where(lanes < 8, e16[hoff + 2 * k],
                                       e16[hoff + 2 * k + 1])
                        g_v[b, sl] = g_v[b, sl] * ev
                return cr
            lax.fori_loop(0, _B, edge, 0)

            pltpu.sync_copy(g_v, acc_sh.at[dst_v], add=True)
            return carry
        lax.fori_loop(0, nchunks, chunk, 0)

        plsc.subcore_barrier()
        row0 = s * rows_per_tile
        pltpu.sync_copy(acc_sh.at[pl.ds(row0, rows_per_tile)],
                        out_hbm.at[c, pl.ds(row0, rows_per_tile)])

    return sc_kernel


def _compact(a):
    hh, cc = a.shape
    eye = jnp.eye(hh, dtype=_F32)
    Ac = (a[:, :, None] * eye[:, None, :]).reshape(hh * cc, hh)
    return jnp.pad(Ac, ((0, 0), (0, 16 - hh)))


def kernel(x, edge_index, W1, a1_src, a1_dst, b1, W2, a2_src, a2_dst, b2,
           W3, a3_src, a3_dst, b3):
    n = x.shape[0]
    loops = jnp.arange(n, dtype=edge_index.dtype)
    src = jnp.concatenate([edge_index[0], loops]).astype(_I32)
    dst = jnp.concatenate([edge_index[1], loops]).astype(_I32)
    e_tot = src.shape[0]
    ept = -(-e_tot // (_NW * _B)) * _B
    e_pad = ept * _NW
    if e_pad > e_tot:
        fill = jnp.full((e_pad - e_tot,), n, _I32)
        src = jnp.concatenate([src, fill])
        dst = jnp.concatenate([dst, fill])
    n_pad = -(-(n + 1) // 1024) * 1024
    xp = jnp.pad(x, ((0, n_pad - n), (0, 0)))

    Ac1s, Ac1d = _compact(a1_src), _compact(a1_dst)
    Ac2s, Ac2d = _compact(a2_src), _compact(a2_dst)
    Ac3s = jnp.pad(_compact(a3_src), ((0, 6), (0, 0)))
    Ac3d = jnp.pad(_compact(a3_dst), ((0, 6), (0, 0)))
    W3p = jnp.pad(W3, ((0, 0), (0, 6)))
    S1 = jnp.repeat(jnp.eye(8, dtype=_F32), 16, axis=1)
    S2 = jnp.repeat(jnp.eye(8, dtype=_F32), 8, axis=1)
    b1r = b1.reshape(1, 128)
    b2r = b2.reshape(1, 64)
    b3r = jnp.pad(b3, (0, 6)).reshape(1, 16)

    grid = (n_pad // _BN,)

    glo, ghi, ad1 = pl.pallas_call(
        _pre_body,
        grid=grid,
        in_specs=[_row_spec(128), _full_spec((128, 128)),
                  _full_spec((128, 16)), _full_spec((128, 16))],
        out_specs=(_row_spec(80), _row_spec(80), _row_spec(16)),
        out_shape=(
            jax.ShapeDtypeStruct((n_pad, 80), _F32),
            jax.ShapeDtypeStruct((n_pad, 80), _F32),
            jax.ShapeDtypeStruct((n_pad, 16), _F32),
        ),
    )(xp, W1, Ac1s, Ac1d)

    accA = _make_sc(n_pad, 64, 16, 0, ept)(glo, ad1, src, dst)
    accB = _make_sc(n_pad, 64, 16, 4, ept)(ghi, ad1, src, dst)

    g2, ad2 = pl.pallas_call(
        _mid2_body,
        grid=grid,
        in_specs=[_acc_spec(80), _acc_spec(80),
                  _full_spec((8, 128)), _full_spec((1, 128)),
                  _full_spec((128, 64)), _full_spec((64, 16)),
                  _full_spec((64, 16))],
        out_specs=(_row_spec(80), _row_spec(16)),
        out_shape=(
            jax.ShapeDtypeStruct((n_pad, 80), _F32),
            jax.ShapeDtypeStruct((n_pad, 16), _F32),
        ),
    )(accA, accB, S1, b1r, W2, Ac2s, Ac2d)

    acc2 = _make_sc(n_pad, 64, 8, 0, ept)(g2, ad2, src, dst)

    g3, ad3 = pl.pallas_call(
        _mid3_body,
        grid=grid,
        in_specs=[_acc_spec(80),
                  _full_spec((8, 64)), _full_spec((1, 64)),
                  _full_spec((64, 16)), _full_spec((16, 16)),
                  _full_spec((16, 16))],
        out_specs=(_row_spec(32), _row_spec(16)),
        out_shape=(
            jax.ShapeDtypeStruct((n_pad, 32), _F32),
            jax.ShapeDtypeStruct((n_pad, 16), _F32),
        ),
    )(acc2, S2, b2r, W3p, Ac3s, Ac3d)

    acc3 = _make_sc(n_pad, 16, 16, 0, ept)(g3, ad3, src, dst)

    out = pl.pallas_call(
        _final_body,
        grid=grid,
        in_specs=[_acc_spec(32), _full_spec((1, 16))],
        out_specs=pl.BlockSpec((_BN, 16), lambda i: (i, 0)),
        out_shape=jax.ShapeDtypeStruct((n_pad, 16), _F32),
    )(acc3, b3r)
    return out[:n, :10]


# speedup vs baseline: 70.3300x; 1.3415x over previous
import functools

import jax
import jax.numpy as jnp
from jax import lax
from jax.experimental import pallas as pl
from jax.experimental.pallas import tpu as pltpu
from jax.experimental.pallas import tpu_sc as plsc

_F32 = jnp.float32
_I32 = jnp.int32

_NW = 32
_B = 128
_ZR = 64
_BN = 1024


def _pre_body(x_ref, W_ref, Acs_ref, Acd_ref,
              glo_ref, ghi_ref, ad_ref):
    h = jnp.dot(x_ref[...], W_ref[...], preferred_element_type=_F32)
    asrc = jnp.dot(h, Acs_ref[...], preferred_element_type=_F32)
    glo_ref[...] = jnp.concatenate([h[:, :64], asrc], axis=1)
    ghi_ref[...] = jnp.concatenate([h[:, 64:], asrc], axis=1)
    ad_ref[...] = jnp.dot(h, Acd_ref[...], preferred_element_type=_F32)


def _mid2_body(accA_ref, accB_ref, S_ref, b_ref, W_ref, Acs_ref, Acd_ref,
               g_ref, ad_ref):
    accA = accA_ref[0] + accA_ref[1]
    accB = accB_ref[0] + accB_ref[1]
    num = jnp.concatenate([accA[:, :64], accB[:, :64]], axis=1)
    recip = 1.0 / (accA[:, 64:72] + 1e-16)
    xn = num * jnp.dot(recip, S_ref[...], preferred_element_type=_F32) + b_ref[...]
    xn = jnp.where(xn > 0, xn, jnp.exp(xn) - 1.0)
    h = jnp.dot(xn, W_ref[...], preferred_element_type=_F32)
    g_ref[...] = jnp.concatenate(
        [h, jnp.dot(h, Acs_ref[...], preferred_element_type=_F32)], axis=1)
    ad_ref[...] = jnp.dot(h, Acd_ref[...], preferred_element_type=_F32)


def _mid3_body(acc_ref, S_ref, b_ref, W_ref, Acs_ref, Acd_ref,
               g_ref, ad_ref):
    acc = acc_ref[0] + acc_ref[1]
    num = acc[:, :64]
    recip = 1.0 / (acc[:, 64:72] + 1e-16)
    xn = num * jnp.dot(recip, S_ref[...], preferred_element_type=_F32) + b_ref[...]
    xn = jnp.where(xn > 0, xn, jnp.exp(xn) - 1.0)
    h = jnp.dot(xn, W_ref[...], preferred_element_type=_F32)
    g_ref[...] = jnp.concatenate(
        [h, jnp.dot(h, Acs_ref[...], preferred_element_type=_F32)], axis=1)
    ad_ref[...] = jnp.dot(h, Acd_ref[...], preferred_element_type=_F32)


def _final_body(acc_ref, b_ref, out_ref):
    acc = acc_ref[0] + acc_ref[1]
    z = acc[:, :16] / (acc[:, 16:17] + 1e-16) + b_ref[...]
    col = lax.broadcasted_iota(_I32, z.shape, 1)
    valid = col < 10
    zm = jnp.where(valid, z, -1e30)
    m = jnp.max(zm, axis=1, keepdims=True)
    ssum = jnp.sum(jnp.where(valid, jnp.exp(z - m), 0.0), axis=1, keepdims=True)
    out_ref[...] = zm - m - jnp.log(ssum)


def _full_spec(shape):
    nd = len(shape)
    return pl.BlockSpec(shape, lambda i, _n=nd: (0,) * _n)


def _row_spec(w):
    return pl.BlockSpec((_BN, w), lambda i: (i, 0))


def _acc_spec(w):
    return pl.BlockSpec((2, _BN, w), lambda i: (0, i, 0))


def _make_sc(n_pad, wm, ch, hoff, ept):
    wr = wm + 16
    nch = wm // 16
    nchunks = ept // _B
    rows_per_tile = n_pad // 16
    mesh = plsc.VectorSubcoreMesh(core_axis_name="c", subcore_axis_name="s")

    @functools.partial(
        pl.kernel,
        out_type=jax.ShapeDtypeStruct((2, n_pad, wr), _F32),
        mesh=mesh,
        compiler_params=pltpu.CompilerParams(use_tc_tiling_on_sc=False),
        scratch_types=[
            pltpu.VMEM((_B,), _I32),
            pltpu.VMEM((_B,), _I32),
            pltpu.VMEM((_B, wr), _F32),
            pltpu.VMEM((_B, 16), _F32),
            pltpu.VMEM((_B,), _I32),
            pltpu.VMEM((_B,), _I32),
            pltpu.VMEM((_B, wr), _F32),
            pltpu.VMEM((_B, 16), _F32),
            pltpu.VMEM((_ZR, wr), _F32),
            pltpu.VMEM_SHARED((n_pad, wr), _F32),
            pltpu.SemaphoreType.DMA,
            pltpu.SemaphoreType.DMA,
        ],
    )
    def sc_kernel(g_hbm, ad_hbm, src_hbm, dst_hbm, out_hbm,
                  src0, dst0, g0, ad0, src1, dst1, g1, ad1, z_v, acc_sh,
                  sem0, sem1):
        c = lax.axis_index("c")
        s = lax.axis_index("s")
        wid = s * 2 + c
        zvec = jnp.zeros((16,), _F32)

        def zero_buf(r, carry):
            for k in range(wr // 16):
                z_v[r, pl.ds(k * 16, 16)] = zvec
            return carry
        lax.fori_loop(0, _ZR, zero_buf, 0)

        def zero_acc(i, carry):
            pltpu.sync_copy(z_v, acc_sh.at[pl.ds(s * rows_per_tile + i * _ZR, _ZR)])
            return carry
        lax.fori_loop(0, rows_per_tile // _ZR, zero_acc, 0)
        plsc.subcore_barrier()

        base_e = wid * ept
        lanes = lax.iota(_I32, 16)

        def issue(chk, sv, dv, gv, av, sem):
            eb = base_e + chk * _B
            pltpu.sync_copy(src_hbm.at[pl.ds(eb, _B)], sv)
            pltpu.sync_copy(dst_hbm.at[pl.ds(eb, _B)], dv)
            pltpu.async_copy(g_hbm.at[sv], gv, sem)
            pltpu.async_copy(ad_hbm.at[dv], av, sem)

        def process(sv, dv, gv, av, sem):
            pltpu.make_async_copy(g_hbm.at[sv], gv, sem).wait()
            pltpu.make_async_copy(ad_hbm.at[dv], av, sem).wait()

            def edge(b, cr):
                al = gv[b, pl.ds(wm, 16)] + av[b, pl.ds(0, 16)]
                e16 = jnp.exp(jnp.maximum(al, 0.2 * al))
                gv[b, pl.ds(wm, 16)] = e16
                for k in range(nch):
                    sl = pl.ds(k * 16, 16)
                    if ch == 16:
                        ev = e16[hoff + k]
                        gv[b, sl] = gv[b, sl] * ev
                    else:
                        ev = jnp.where(lanes < 8, e16[hoff + 2 * k],
                                       e16[hoff + 2 * k + 1])
                        gv[b, sl] = gv[b, sl] * ev
                return cr
            lax.fori_loop(0, _B, edge, 0)

            pltpu.sync_copy(gv, acc_sh.at[dv], add=True)

        issue(0, src0, dst0, g0, ad0, sem0)

        def pair(p, carry):
            chk0 = 2 * p

            @pl.when(chk0 + 1 < nchunks)
            def _():
                issue(chk0 + 1, src1, dst1, g1, ad1, sem1)
            process(src0, dst0, g0, ad0, sem0)

            @pl.when(chk0 + 2 < nchunks)
            def _():
                issue(chk0 + 2, src0, dst0, g0, ad0, sem0)

            @pl.when(chk0 + 1 < nchunks)
            def _():
                process(src1, dst1, g1, ad1, sem1)
            return carry
        lax.fori_loop(0, (nchunks + 1) // 2, pair, 0)

        plsc.subcore_barrier()
        row0 = s * rows_per_tile
        pltpu.sync_copy(acc_sh.at[pl.ds(row0, rows_per_tile)],
                        out_hbm.at[c, pl.ds(row0, rows_per_tile)])

    return sc_kernel


def _compact(a):
    hh, cc = a.shape
    eye = jnp.eye(hh, dtype=_F32)
    Ac = (a[:, :, None] * eye[:, None, :]).reshape(hh * cc, hh)
    return jnp.pad(Ac, ((0, 0), (0, 16 - hh)))


def kernel(x, edge_index, W1, a1_src, a1_dst, b1, W2, a2_src, a2_dst, b2,
           W3, a3_src, a3_dst, b3):
    n = x.shape[0]
    loops = jnp.arange(n, dtype=edge_index.dtype)
    src = jnp.concatenate([edge_index[0], loops]).astype(_I32)
    dst = jnp.concatenate([edge_index[1], loops]).astype(_I32)
    e_tot = src.shape[0]
    ept = -(-e_tot // (_NW * _B)) * _B
    e_pad = ept * _NW
    if e_pad > e_tot:
        fill = jnp.full((e_pad - e_tot,), n, _I32)
        src = jnp.concatenate([src, fill])
        dst = jnp.concatenate([dst, fill])
    n_pad = -(-(n + 1) // 1024) * 1024
    xp = jnp.pad(x, ((0, n_pad - n), (0, 0)))

    Ac1s, Ac1d = _compact(a1_src), _compact(a1_dst)
    Ac2s, Ac2d = _compact(a2_src), _compact(a2_dst)
    Ac3s = jnp.pad(_compact(a3_src), ((0, 6), (0, 0)))
    Ac3d = jnp.pad(_compact(a3_dst), ((0, 6), (0, 0)))
    W3p = jnp.pad(W3, ((0, 0), (0, 6)))
    S1 = jnp.repeat(jnp.eye(8, dtype=_F32), 16, axis=1)
    S2 = jnp.repeat(jnp.eye(8, dtype=_F32), 8, axis=1)
    b1r = b1.reshape(1, 128)
    b2r = b2.reshape(1, 64)
    b3r = jnp.pad(b3, (0, 6)).reshape(1, 16)

    grid = (n_pad // _BN,)

    glo, ghi, ad1 = pl.pallas_call(
        _pre_body,
        grid=grid,
        in_specs=[_row_spec(128), _full_spec((128, 128)),
                  _full_spec((128, 16)), _full_spec((128, 16))],
        out_specs=(_row_spec(80), _row_spec(80), _row_spec(16)),
        out_shape=(
            jax.ShapeDtypeStruct((n_pad, 80), _F32),
            jax.ShapeDtypeStruct((n_pad, 80), _F32),
            jax.ShapeDtypeStruct((n_pad, 16), _F32),
        ),
    )(xp, W1, Ac1s, Ac1d)

    accA = _make_sc(n_pad, 64, 16, 0, ept)(glo, ad1, src, dst)
    accB = _make_sc(n_pad, 64, 16, 4, ept)(ghi, ad1, src, dst)

    g2, ad2 = pl.pallas_call(
        _mid2_body,
        grid=grid,
        in_specs=[_acc_spec(80), _acc_spec(80),
                  _full_spec((8, 128)), _full_spec((1, 128)),
                  _full_spec((128, 64)), _full_spec((64, 16)),
                  _full_spec((64, 16))],
        out_specs=(_row_spec(80), _row_spec(16)),
        out_shape=(
            jax.ShapeDtypeStruct((n_pad, 80), _F32),
            jax.ShapeDtypeStruct((n_pad, 16), _F32),
        ),
    )(accA, accB, S1, b1r, W2, Ac2s, Ac2d)

    acc2 = _make_sc(n_pad, 64, 8, 0, ept)(g2, ad2, src, dst)

    g3, ad3 = pl.pallas_call(
        _mid3_body,
        grid=grid,
        in_specs=[_acc_spec(80),
                  _full_spec((8, 64)), _full_spec((1, 64)),
                  _full_spec((64, 16)), _full_spec((16, 16)),
                  _full_spec((16, 16))],
        out_specs=(_row_spec(32), _row_spec(16)),
        out_shape=(
            jax.ShapeDtypeStruct((n_pad, 32), _F32),
            jax.ShapeDtypeStruct((n_pad, 16), _F32),
        ),
    )(acc2, S2, b2r, W3p, Ac3s, Ac3d)

    acc3 = _make_sc(n_pad, 16, 16, 0, ept)(g3, ad3, src, dst)

    out = pl.pallas_call(
        _final_body,
        grid=grid,
        in_specs=[_acc_spec(32), _full_spec((1, 16))],
        out_specs=pl.BlockSpec((_BN, 16), lambda i: (i, 0)),
        out_shape=jax.ShapeDtypeStruct((n_pad, 16), _F32),
    )(acc3, b3r)
    return out[:n, :10]
